# Initial kernel scaffold; baseline (speedup 1.0000x reference)
import jax
import jax.numpy as jnp
from jax.experimental import pallas as pl


def kernel(pos, edge_index, params):
    raise NotImplementedError("write your pallas kernel here")


# speedup vs baseline: 11.4758x; 11.4758x over previous
import functools

import jax
import jax.numpy as jnp
from jax import lax
from jax.experimental import pallas as pl
from jax.experimental.pallas import tpu as pltpu
from jax.experimental.pallas import tpu_sc as plsc

N = 10000
E_RAW = 320000
NW = 32
EPB = 120
NPB = 88
E_PAD = NW * NPB * EPB
ACC_ROWS = 10240
ZB = 64
MR = 400
_MESH = dict(core_axis_name="c", subcore_axis_name="s",
             num_cores=2, num_subcores=16)


@functools.lru_cache(maxsize=None)
def _make_deg():
    return functools.partial(
        pl.kernel,
        out_type=jax.ShapeDtypeStruct((2, ACC_ROWS, 128), jnp.float32),
        mesh=plsc.VectorSubcoreMesh(**_MESH),
        scratch_types=[
            pltpu.VMEM((NPB, EPB), jnp.int32),
            pltpu.VMEM((EPB, 128), jnp.float32),
            pltpu.VMEM((ZB, 128), jnp.float32),
            pltpu.VMEM_SHARED((ACC_ROWS, 128), jnp.float32),
        ],
    )(_deg_body)


def _deg_body(dst_hbm, ones_hbm, zeros_hbm, out_hbm, dstv, onesv, zv, acc):
    c = lax.axis_index("c")
    s = lax.axis_index("s")
    wid = c * 16 + s
    pltpu.sync_copy(dst_hbm.at[wid], dstv)
    pltpu.sync_copy(ones_hbm, onesv)
    pltpu.sync_copy(zeros_hbm, zv)
    for z in range(640 // ZB):
        pltpu.sync_copy(zv, acc.at[pl.ds(s * 640 + z * ZB, ZB)])
    plsc.subcore_barrier()

    def body(b, carry):
        pltpu.sync_copy(onesv, acc.at[dstv.at[b]], add=True)
        return carry

    lax.fori_loop(0, NPB, body, 0)
    plsc.subcore_barrier()
    pltpu.sync_copy(acc.at[pl.ds(s * 640, 640)],
                    out_hbm.at[c, pl.ds(s * 640, 640)])


@functools.lru_cache(maxsize=None)
def _make_prop(G, C):
    @functools.partial(
        pl.kernel,
        out_type=jax.ShapeDtypeStruct((2, G, ACC_ROWS, C), jnp.float32),
        mesh=plsc.VectorSubcoreMesh(**_MESH),
        scratch_types=[
            pltpu.VMEM((2, EPB), jnp.int32),
            pltpu.VMEM((2, EPB), jnp.int32),
            pltpu.VMEM((2, EPB), jnp.int32),
            pltpu.VMEM((2, EPB), jnp.int32),
            pltpu.VMEM((EPB, C), jnp.float32),
            pltpu.VMEM((EPB, C), jnp.float32),
            pltpu.VMEM((ZB, C), jnp.float32),
            pltpu.VMEM_SHARED((ACC_ROWS, C), jnp.float32),
            pltpu.SemaphoreType.DMA,
            pltpu.SemaphoreType.DMA,
            pltpu.SemaphoreType.DMA,
            pltpu.SemaphoreType.DMA,
            pltpu.SemaphoreType.DMA,
            pltpu.SemaphoreType.DMA,
        ],
    )
    def _prop(idx_hbm, zeros_hbm, hp_hbm, out_hbm,
              i0, i1, i2, i3, msg0, msg1, zv, acc,
              s0, s1, s2, s3, gsem, ssem):
        c = lax.axis_index("c")
        s = lax.axis_index("s")
        wid = c * 16 + s
        islot = (i0, i1, i2, i3)
        isem = (s0, s1, s2, s3)
        msg = (msg0, msg1)
        my_idx = idx_hbm.at[wid]
        pltpu.sync_copy(zeros_hbm, zv)

        for g in range(G):
            for z in range(640 // ZB):
                pltpu.sync_copy(zv, acc.at[pl.ds(s * 640 + z * ZB, ZB)])
            plsc.subcore_barrier()

            table = hp_hbm.at[g]

            for b in range(3):
                pltpu.async_copy(my_idx.at[b], islot[b], isem[b])
            pltpu.make_async_copy(my_idx.at[0], islot[0], isem[0]).wait()
            pltpu.async_copy(table.at[islot[0].at[0]], msg[0], gsem)

            def stage(b, u):
                jn = (u + 1) % 4
                jp = (u + 3) % 4

                @pl.when(b + 1 < NPB)
                def _():
                    pltpu.make_async_copy(
                        my_idx.at[b + 1], islot[jn], isem[jn]).wait()
                pltpu.make_async_copy(
                    table.at[islot[u].at[0]], msg[u % 2], gsem).wait()

                @pl.when(b >= 1)
                def _():
                    pltpu.make_async_copy(
                        msg[(u + 1) % 2], acc.at[islot[jp].at[1]],
                        ssem).wait()

                @pl.when(b + 3 < NPB)
                def _():
                    pltpu.async_copy(my_idx.at[b + 3], islot[jp], isem[jp])

                @pl.when(b + 1 < NPB)
                def _():
                    pltpu.async_copy(
                        table.at[islot[jn].at[0]], msg[(u + 1) % 2], gsem)

                pltpu.async_copy(
                    msg[u % 2], acc.at[islot[u].at[1]], ssem, add=True)

            def body4(i, carry):
                b0 = i * 4
                for u in range(4):
                    stage(b0 + u, u)
                return carry

            lax.fori_loop(0, NPB // 4, body4, 0)
            pltpu.make_async_copy(
                msg[(NPB - 1) % 2], acc.at[islot[(NPB - 1) % 4].at[1]],
                ssem).wait()
            plsc.subcore_barrier()
            pltpu.sync_copy(acc.at[pl.ds(s * 640, 640)],
                            out_hbm.at[c, g, pl.ds(s * 640, 640)])
            plsc.subcore_barrier()

    return _prop


_GRID = N // MR


def _affine_consts(st_ref, gamma_ref, beta_ref):
    mu = st_ref[0:1, :] / N
    var = st_ref[1:2, :] / N - mu * mu
    scale = gamma_ref[...] * lax.rsqrt(var + 1e-5)
    shift = beta_ref[...] - mu * scale
    return scale, shift


def _accum_stats(st_ref, y):
    @pl.when(pl.program_id(0) == 0)
    def _():
        st_ref[...] = jnp.zeros_like(st_ref)

    st_ref[0:1, :] += jnp.sum(y, axis=0, keepdims=True)
    st_ref[1:2, :] += jnp.sum(y * y, axis=0, keepdims=True)


def _dot(x, w):
    return jnp.dot(x, w, preferred_element_type=jnp.float32)


def _dinv_kernel(deg_ref, o_ref):
    d = deg_ref[0, :, 0:1] + deg_ref[1, :, 0:1]
    o_ref[...] = lax.rsqrt(d)


def _dinv(deg):
    return pl.pallas_call(
        _dinv_kernel,
        grid=(_GRID,),
        in_specs=[pl.BlockSpec((2, MR, 128), lambda m: (0, m, 0))],
        out_specs=pl.BlockSpec((MR, 1), lambda m: (m, 0)),
        out_shape=jax.ShapeDtypeStruct((N, 1), jnp.float32),
    )(deg)


def _conv1_kernel(x_ref, w_ref, dinv_ref, o_ref):
    o_ref[0] = _dot(x_ref[...], w_ref[...]) * dinv_ref[...]


def _conv1_pre(pos, w, dinv):
    kin, kout = w.shape
    return pl.pallas_call(
        _conv1_kernel,
        grid=(_GRID,),
        in_specs=[
            pl.BlockSpec((MR, kin), lambda m: (m, 0)),
            pl.BlockSpec((kin, kout), lambda m: (0, 0)),
            pl.BlockSpec((MR, 1), lambda m: (m, 0)),
        ],
        out_specs=pl.BlockSpec((1, MR, kout), lambda m: (0, m, 0)),
        out_shape=jax.ShapeDtypeStruct((1, N, kout), jnp.float32),
    )(pos, w, dinv)


def _convpre_kernel(G, C, y_ref, st_ref, g_ref, be_ref, w_ref, dinv_ref, o_ref):
    scale, shift = _affine_consts(st_ref, g_ref, be_ref)
    z = y_ref[...] * scale + shift
    h = _dot(z, w_ref[...]) * dinv_ref[...]
    for g in range(G):
        o_ref[g] = h[:, g * C:(g + 1) * C]


def _conv_pre(y, st, gamma, beta, w, dinv, G, C):
    kin = w.shape[0]
    return pl.pallas_call(
        functools.partial(_convpre_kernel, G, C),
        grid=(_GRID,),
        in_specs=[
            pl.BlockSpec((MR, kin), lambda m: (m, 0)),
            pl.BlockSpec((8, kin), lambda m: (0, 0)),
            pl.BlockSpec((1, kin), lambda m: (0, 0)),
            pl.BlockSpec((1, kin), lambda m: (0, 0)),
            pl.BlockSpec((kin, G * C), lambda m: (0, 0)),
            pl.BlockSpec((MR, 1), lambda m: (m, 0)),
        ],
        out_specs=pl.BlockSpec((G, MR, C), lambda m: (0, m, 0)),
        out_shape=jax.ShapeDtypeStruct((G, N, C), jnp.float32),
    )(y, st, gamma, beta, w, dinv)


def _mlpfirst_kernel(G, s_ref, dinv_ref, bc_ref, w_ref, b_ref, y_ref, st_ref):
    parts = [s_ref[0, g] + s_ref[1, g] for g in range(G)]
    x = parts[0] if G == 1 else jnp.concatenate(parts, axis=1)
    x = x * dinv_ref[...] + bc_ref[...]
    y = jnp.maximum(_dot(x, w_ref[...]) + b_ref[...], 0.0)
    y_ref[...] = y
    _accum_stats(st_ref, y)


def _mlp_first(sagg, dinv, bconv, w, b, G, C):
    kin, kout = w.shape
    return pl.pallas_call(
        functools.partial(_mlpfirst_kernel, G),
        grid=(_GRID,),
        in_specs=[
            pl.BlockSpec((2, G, MR, C), lambda m: (0, 0, m, 0)),
            pl.BlockSpec((MR, 1), lambda m: (m, 0)),
            pl.BlockSpec((1, kin), lambda m: (0, 0)),
            pl.BlockSpec((kin, kout), lambda m: (0, 0)),
            pl.BlockSpec((1, kout), lambda m: (0, 0)),
        ],
        out_specs=[
            pl.BlockSpec((MR, kout), lambda m: (m, 0)),
            pl.BlockSpec((8, kout), lambda m: (0, 0)),
        ],
        out_shape=[
            jax.ShapeDtypeStruct((N, kout), jnp.float32),
            jax.ShapeDtypeStruct((8, kout), jnp.float32),
        ],
    )(sagg, dinv, bconv, w, b)


def _mlpmid_kernel(y_ref, st_in_ref, g_ref, be_ref, w_ref, b_ref,
                   y_ref_o, st_ref_o):
    scale, shift = _affine_consts(st_in_ref, g_ref, be_ref)
    z = y_ref[...] * scale + shift
    y = jnp.maximum(_dot(z, w_ref[...]) + b_ref[...], 0.0)
    y_ref_o[...] = y
    _accum_stats(st_ref_o, y)


def _mlp_mid(y, st, gamma, beta, w, b):
    kin, kout = w.shape
    return pl.pallas_call(
        _mlpmid_kernel,
        grid=(_GRID,),
        in_specs=[
            pl.BlockSpec((MR, kin), lambda m: (m, 0)),
            pl.BlockSpec((8, kin), lambda m: (0, 0)),
            pl.BlockSpec((1, kin), lambda m: (0, 0)),
            pl.BlockSpec((1, kin), lambda m: (0, 0)),
            pl.BlockSpec((kin, kout), lambda m: (0, 0)),
            pl.BlockSpec((1, kout), lambda m: (0, 0)),
        ],
        out_specs=[
            pl.BlockSpec((MR, kout), lambda m: (m, 0)),
            pl.BlockSpec((8, kout), lambda m: (0, 0)),
        ],
        out_shape=[
            jax.ShapeDtypeStruct((N, kout), jnp.float32),
            jax.ShapeDtypeStruct((8, kout), jnp.float32),
        ],
    )(y, st, gamma, beta, w, b)


def _head_kernel(y_ref, st_ref, g_ref, be_ref,
                 w1_ref, b1_ref, w2_ref, b2_ref, w3_ref, b3_ref, o_ref):
    scale, shift = _affine_consts(st_ref, g_ref, be_ref)
    z = y_ref[...] * scale + shift
    a = _dot(z, w1_ref[...]) + b1_ref[...]
    a = _dot(jnp.maximum(a, 0.0), w2_ref[...]) + b2_ref[...]
    a = _dot(a, w3_ref[...]) + b3_ref[...]
    m = jnp.max(a, axis=1, keepdims=True)
    e = jnp.exp(a - m)
    o_ref[...] = a - m - jnp.log(jnp.sum(e, axis=1, keepdims=True))


def _head(y, st, gamma, beta, w1, b1, w2, b2, w3, b3):
    k1 = w1.shape[0]
    d1, d2, d3 = w1.shape[1], w2.shape[1], w3.shape[1]
    return pl.pallas_call(
        _head_kernel,
        grid=(_GRID,),
        in_specs=[
            pl.BlockSpec((MR, k1), lambda m: (m, 0)),
            pl.BlockSpec((8, k1), lambda m: (0, 0)),
            pl.BlockSpec((1, k1), lambda m: (0, 0)),
            pl.BlockSpec((1, k1), lambda m: (0, 0)),
            pl.BlockSpec((k1, d1), lambda m: (0, 0)),
            pl.BlockSpec((1, d1), lambda m: (0, 0)),
            pl.BlockSpec((d1, d2), lambda m: (0, 0)),
            pl.BlockSpec((1, d2), lambda m: (0, 0)),
            pl.BlockSpec((d2, d3), lambda m: (0, 0)),
            pl.BlockSpec((1, d3), lambda m: (0, 0)),
        ],
        out_specs=pl.BlockSpec((MR, d3), lambda m: (m, 0)),
        out_shape=jax.ShapeDtypeStruct((N, d3), jnp.float32),
    )(y, st, gamma, beta, w1, b1, w2, b2, w3, b3)


def _row(v):
    return v.reshape(1, -1)


def kernel(pos, edge_index, params):
    loop = jnp.arange(N, dtype=jnp.int32)
    pad = E_PAD - E_RAW - N
    padr = jnp.arange(pad, dtype=jnp.int32)
    srcs = jnp.concatenate([edge_index[0], loop, padr % N])
    dsts = jnp.concatenate([edge_index[1], loop, N + padr % (ACC_ROWS - N)])
    src3 = srcs.reshape(NW, NPB, EPB)
    dst3 = dsts.reshape(NW, NPB, EPB)
    idx4 = jnp.stack([src3, dst3], axis=2)
    ones128 = jnp.ones((EPB, 128), jnp.float32)
    zeros128 = jnp.zeros((ZB, 128), jnp.float32)

    deg = _make_deg()(dst3, ones128, zeros128)
    dinv = _dinv(deg)

    convs = [("conv1", "mlp1", 1, 128), ("conv2", "mlp2", 1, 128),
             ("conv3", "mlp3", 1, 128), ("conv4", "mlp4", 2, 128),
             ("conv5", "mlp5", 4, 128)]

    y, st, gam, bet = pos, None, None, None
    for cname, mname, G, C in convs:
        cw = params[cname]["W"]
        cb = params[cname]["b"]
        layers = params[mname]
        w1 = layers[0]["W"]
        if st is None:
            cw = jnp.pad(cw, ((0, 0), (0, C - cw.shape[1])))
            cb = jnp.pad(cb, (0, C - cb.shape[0]))
            w1 = jnp.pad(w1, ((0, C - w1.shape[0]), (0, 0)))
            hp = _conv1_pre(pos, cw, dinv)
        else:
            hp = _conv_pre(y, st, gam, bet, cw, dinv, G, C)
        zeros_c = jnp.zeros((ZB, C), jnp.float32)
        sagg = _make_prop(G, C)(idx4, zeros_c, hp)
        y, st = _mlp_first(sagg, dinv, _row(cb), w1,
                           _row(layers[0]["b"]), G, C)
        gam, bet = _row(layers[0]["gamma"]), _row(layers[0]["beta"])
        for l in layers[1:]:
            y, st = _mlp_mid(y, st, gam, bet, l["W"], _row(l["b"]))
            gam, bet = _row(l["gamma"]), _row(l["beta"])

    p1, p2, p3 = params["lin1"], params["lin2"], params["lin3"]
    return _head(y, st, gam, bet, p1["W"], _row(p1["b"]),
                 p2["W"], _row(p2["b"]), p3["W"], _row(p3["b"]))


# speedup vs baseline: 13.1111x; 1.1425x over previous
import functools

import jax
import jax.numpy as jnp
from jax import lax
from jax.experimental import pallas as pl
from jax.experimental.pallas import tpu as pltpu
from jax.experimental.pallas import tpu_sc as plsc

N = 10000
E_RAW = 320000
NW = 32
EPB = 120
NPB = 88
E_PAD = NW * NPB * EPB
ACC_ROWS = 10240
ZB = 64
MR = 400
_MESH = dict(core_axis_name="c", subcore_axis_name="s",
             num_cores=2, num_subcores=16)


@functools.lru_cache(maxsize=None)
def _make_deg():
    return functools.partial(
        pl.kernel,
        out_type=jax.ShapeDtypeStruct((2, ACC_ROWS, 128), jnp.float32),
        mesh=plsc.VectorSubcoreMesh(**_MESH),
        scratch_types=[
            pltpu.VMEM((NPB, EPB), jnp.int32),
            pltpu.VMEM((EPB, 128), jnp.float32),
            pltpu.VMEM((ZB, 128), jnp.float32),
            pltpu.VMEM_SHARED((ACC_ROWS, 128), jnp.float32),
        ],
    )(_deg_body)


def _deg_body(dst_hbm, ones_hbm, zeros_hbm, out_hbm, dstv, onesv, zv, acc):
    c = lax.axis_index("c")
    s = lax.axis_index("s")
    wid = c * 16 + s
    pltpu.sync_copy(dst_hbm.at[wid], dstv)
    pltpu.sync_copy(ones_hbm, onesv)
    pltpu.sync_copy(zeros_hbm, zv)
    for z in range(640 // ZB):
        pltpu.sync_copy(zv, acc.at[pl.ds(s * 640 + z * ZB, ZB)])
    plsc.subcore_barrier()

    def body(b, carry):
        pltpu.sync_copy(onesv, acc.at[dstv.at[b]], add=True)
        return carry

    lax.fori_loop(0, NPB, body, 0)
    plsc.subcore_barrier()
    pltpu.sync_copy(acc.at[pl.ds(s * 640, 640)],
                    out_hbm.at[c, pl.ds(s * 640, 640)])


@functools.lru_cache(maxsize=None)
def _make_prop(G, C):
    @functools.partial(
        pl.kernel,
        out_type=jax.ShapeDtypeStruct((2, G, ACC_ROWS, C), jnp.float32),
        mesh=plsc.VectorSubcoreMesh(**_MESH),
        scratch_types=[
            pltpu.VMEM((2, EPB), jnp.int32),
            pltpu.VMEM((2, EPB), jnp.int32),
            pltpu.VMEM((2, EPB), jnp.int32),
            pltpu.VMEM((2, EPB), jnp.int32),
            pltpu.VMEM((EPB, C), jnp.float32),
            pltpu.VMEM((EPB, C), jnp.float32),
            pltpu.VMEM((ZB, C), jnp.float32),
            pltpu.VMEM_SHARED((ACC_ROWS, C), jnp.float32),
            pltpu.SemaphoreType.DMA,
            pltpu.SemaphoreType.DMA,
            pltpu.SemaphoreType.DMA,
            pltpu.SemaphoreType.DMA,
            pltpu.SemaphoreType.DMA,
            pltpu.SemaphoreType.DMA,
            pltpu.SemaphoreType.DMA,
            pltpu.SemaphoreType.DMA,
        ],
    )
    def _prop(idx_hbm, zeros_hbm, hp_hbm, out_hbm,
              i0, i1, i2, i3, msg0, msg1, zv, acc,
              s0, s1, s2, s3, g0, g1, t0, t1):
        c = lax.axis_index("c")
        s = lax.axis_index("s")
        wid = c * 16 + s
        islot = (i0, i1, i2, i3)
        isem = (s0, s1, s2, s3)
        msg = (msg0, msg1)
        gsem = (g0, g1)
        ssem = (t0, t1)
        my_idx = idx_hbm.at[wid]
        pltpu.sync_copy(zeros_hbm, zv)

        for g in range(G):
            for z in range(640 // ZB):
                pltpu.sync_copy(zv, acc.at[pl.ds(s * 640 + z * ZB, ZB)])
            plsc.subcore_barrier()

            table = hp_hbm.at[g]

            for b in range(3):
                pltpu.async_copy(my_idx.at[b], islot[b], isem[b])
            pltpu.make_async_copy(my_idx.at[0], islot[0], isem[0]).wait()
            pltpu.async_copy(table.at[islot[0].at[0]], msg[0], gsem[0])

            def stage(b, u):
                jn = (u + 1) % 4
                jp = (u + 3) % 4

                @pl.when(b + 1 < NPB)
                def _():
                    pltpu.make_async_copy(
                        my_idx.at[b + 1], islot[jn], isem[jn]).wait()

                    @pl.when(b >= 1)
                    def _():
                        pltpu.make_async_copy(
                            msg[(u + 1) % 2], acc.at[islot[jp].at[1]],
                            ssem[(u + 1) % 2]).wait()
                    pltpu.async_copy(
                        table.at[islot[jn].at[0]], msg[(u + 1) % 2],
                        gsem[(u + 1) % 2])

                pltpu.make_async_copy(
                    table.at[islot[u].at[0]], msg[u % 2], gsem[u % 2]).wait()
                pltpu.async_copy(
                    msg[u % 2], acc.at[islot[u].at[1]], ssem[u % 2],
                    add=True)

                @pl.when(b + 3 < NPB)
                def _():
                    pltpu.async_copy(my_idx.at[b + 3], islot[jp], isem[jp])

            def body4(i, carry):
                b0 = i * 4
                for u in range(4):
                    stage(b0 + u, u)
                return carry

            lax.fori_loop(0, NPB // 4, body4, 0)
            pltpu.make_async_copy(
                msg[0], acc.at[islot[(NPB - 2) % 4].at[1]], ssem[0]).wait()
            pltpu.make_async_copy(
                msg[1], acc.at[islot[(NPB - 1) % 4].at[1]], ssem[1]).wait()
            plsc.subcore_barrier()
            pltpu.sync_copy(acc.at[pl.ds(s * 640, 640)],
                            out_hbm.at[c, g, pl.ds(s * 640, 640)])
            plsc.subcore_barrier()

    return _prop


_GRID = N // MR


def _affine_consts(st_ref, gamma_ref, beta_ref):
    mu = st_ref[0:1, :] / N
    var = st_ref[1:2, :] / N - mu * mu
    scale = gamma_ref[...] * lax.rsqrt(var + 1e-5)
    shift = beta_ref[...] - mu * scale
    return scale, shift


def _accum_stats(st_ref, y):
    @pl.when(pl.program_id(0) == 0)
    def _():
        st_ref[...] = jnp.zeros_like(st_ref)

    st_ref[0:1, :] += jnp.sum(y, axis=0, keepdims=True)
    st_ref[1:2, :] += jnp.sum(y * y, axis=0, keepdims=True)


def _dot(x, w):
    return jnp.dot(x, w, preferred_element_type=jnp.float32)


def _dinv_kernel(deg_ref, o_ref):
    d = deg_ref[0, :, 0:1] + deg_ref[1, :, 0:1]
    o_ref[...] = lax.rsqrt(d)


def _dinv(deg):
    return pl.pallas_call(
        _dinv_kernel,
        grid=(_GRID,),
        in_specs=[pl.BlockSpec((2, MR, 128), lambda m: (0, m, 0))],
        out_specs=pl.BlockSpec((MR, 1), lambda m: (m, 0)),
        out_shape=jax.ShapeDtypeStruct((N, 1), jnp.float32),
    )(deg)


def _conv1_kernel(x_ref, w_ref, dinv_ref, o_ref):
    o_ref[0] = _dot(x_ref[...], w_ref[...]) * dinv_ref[...]


def _conv1_pre(pos, w, dinv):
    kin, kout = w.shape
    return pl.pallas_call(
        _conv1_kernel,
        grid=(_GRID,),
        in_specs=[
            pl.BlockSpec((MR, kin), lambda m: (m, 0)),
            pl.BlockSpec((kin, kout), lambda m: (0, 0)),
            pl.BlockSpec((MR, 1), lambda m: (m, 0)),
        ],
        out_specs=pl.BlockSpec((1, MR, kout), lambda m: (0, m, 0)),
        out_shape=jax.ShapeDtypeStruct((1, N, kout), jnp.float32),
    )(pos, w, dinv)


def _convpre_kernel(G, C, y_ref, st_ref, g_ref, be_ref, w_ref, dinv_ref, o_ref):
    scale, shift = _affine_consts(st_ref, g_ref, be_ref)
    z = y_ref[...] * scale + shift
    h = _dot(z, w_ref[...]) * dinv_ref[...]
    for g in range(G):
        o_ref[g] = h[:, g * C:(g + 1) * C]


def _conv_pre(y, st, gamma, beta, w, dinv, G, C):
    kin = w.shape[0]
    return pl.pallas_call(
        functools.partial(_convpre_kernel, G, C),
        grid=(_GRID,),
        in_specs=[
            pl.BlockSpec((MR, kin), lambda m: (m, 0)),
            pl.BlockSpec((8, kin), lambda m: (0, 0)),
            pl.BlockSpec((1, kin), lambda m: (0, 0)),
            pl.BlockSpec((1, kin), lambda m: (0, 0)),
            pl.BlockSpec((kin, G * C), lambda m: (0, 0)),
            pl.BlockSpec((MR, 1), lambda m: (m, 0)),
        ],
        out_specs=pl.BlockSpec((G, MR, C), lambda m: (0, m, 0)),
        out_shape=jax.ShapeDtypeStruct((G, N, C), jnp.float32),
    )(y, st, gamma, beta, w, dinv)


def _mlpfirst_kernel(G, s_ref, dinv_ref, bc_ref, w_ref, b_ref, y_ref, st_ref):
    parts = [s_ref[0, g] + s_ref[1, g] for g in range(G)]
    x = parts[0] if G == 1 else jnp.concatenate(parts, axis=1)
    x = x * dinv_ref[...] + bc_ref[...]
    y = jnp.maximum(_dot(x, w_ref[...]) + b_ref[...], 0.0)
    y_ref[...] = y
    _accum_stats(st_ref, y)


def _mlp_first(sagg, dinv, bconv, w, b, G, C):
    kin, kout = w.shape
    return pl.pallas_call(
        functools.partial(_mlpfirst_kernel, G),
        grid=(_GRID,),
        in_specs=[
            pl.BlockSpec((2, G, MR, C), lambda m: (0, 0, m, 0)),
            pl.BlockSpec((MR, 1), lambda m: (m, 0)),
            pl.BlockSpec((1, kin), lambda m: (0, 0)),
            pl.BlockSpec((kin, kout), lambda m: (0, 0)),
            pl.BlockSpec((1, kout), lambda m: (0, 0)),
        ],
        out_specs=[
            pl.BlockSpec((MR, kout), lambda m: (m, 0)),
            pl.BlockSpec((8, kout), lambda m: (0, 0)),
        ],
        out_shape=[
            jax.ShapeDtypeStruct((N, kout), jnp.float32),
            jax.ShapeDtypeStruct((8, kout), jnp.float32),
        ],
    )(sagg, dinv, bconv, w, b)


def _mlpmid_kernel(y_ref, st_in_ref, g_ref, be_ref, w_ref, b_ref,
                   y_ref_o, st_ref_o):
    scale, shift = _affine_consts(st_in_ref, g_ref, be_ref)
    z = y_ref[...] * scale + shift
    y = jnp.maximum(_dot(z, w_ref[...]) + b_ref[...], 0.0)
    y_ref_o[...] = y
    _accum_stats(st_ref_o, y)


def _mlp_mid(y, st, gamma, beta, w, b):
    kin, kout = w.shape
    return pl.pallas_call(
        _mlpmid_kernel,
        grid=(_GRID,),
        in_specs=[
            pl.BlockSpec((MR, kin), lambda m: (m, 0)),
            pl.BlockSpec((8, kin), lambda m: (0, 0)),
            pl.BlockSpec((1, kin), lambda m: (0, 0)),
            pl.BlockSpec((1, kin), lambda m: (0, 0)),
            pl.BlockSpec((kin, kout), lambda m: (0, 0)),
            pl.BlockSpec((1, kout), lambda m: (0, 0)),
        ],
        out_specs=[
            pl.BlockSpec((MR, kout), lambda m: (m, 0)),
            pl.BlockSpec((8, kout), lambda m: (0, 0)),
        ],
        out_shape=[
            jax.ShapeDtypeStruct((N, kout), jnp.float32),
            jax.ShapeDtypeStruct((8, kout), jnp.float32),
        ],
    )(y, st, gamma, beta, w, b)


def _head_kernel(y_ref, st_ref, g_ref, be_ref,
                 w1_ref, b1_ref, w2_ref, b2_ref, w3_ref, b3_ref, o_ref):
    scale, shift = _affine_consts(st_ref, g_ref, be_ref)
    z = y_ref[...] * scale + shift
    a = _dot(z, w1_ref[...]) + b1_ref[...]
    a = _dot(jnp.maximum(a, 0.0), w2_ref[...]) + b2_ref[...]
    a = _dot(a, w3_ref[...]) + b3_ref[...]
    m = jnp.max(a, axis=1, keepdims=True)
    e = jnp.exp(a - m)
    o_ref[...] = a - m - jnp.log(jnp.sum(e, axis=1, keepdims=True))


def _head(y, st, gamma, beta, w1, b1, w2, b2, w3, b3):
    k1 = w1.shape[0]
    d1, d2, d3 = w1.shape[1], w2.shape[1], w3.shape[1]
    return pl.pallas_call(
        _head_kernel,
        grid=(_GRID,),
        in_specs=[
            pl.BlockSpec((MR, k1), lambda m: (m, 0)),
            pl.BlockSpec((8, k1), lambda m: (0, 0)),
            pl.BlockSpec((1, k1), lambda m: (0, 0)),
            pl.BlockSpec((1, k1), lambda m: (0, 0)),
            pl.BlockSpec((k1, d1), lambda m: (0, 0)),
            pl.BlockSpec((1, d1), lambda m: (0, 0)),
            pl.BlockSpec((d1, d2), lambda m: (0, 0)),
            pl.BlockSpec((1, d2), lambda m: (0, 0)),
            pl.BlockSpec((d2, d3), lambda m: (0, 0)),
            pl.BlockSpec((1, d3), lambda m: (0, 0)),
        ],
        out_specs=pl.BlockSpec((MR, d3), lambda m: (m, 0)),
        out_shape=jax.ShapeDtypeStruct((N, d3), jnp.float32),
    )(y, st, gamma, beta, w1, b1, w2, b2, w3, b3)


def _row(v):
    return v.reshape(1, -1)


def kernel(pos, edge_index, params):
    loop = jnp.arange(N, dtype=jnp.int32)
    pad = E_PAD - E_RAW - N
    padr = jnp.arange(pad, dtype=jnp.int32)
    srcs = jnp.concatenate([edge_index[0], loop, padr % N])
    dsts = jnp.concatenate([edge_index[1], loop, N + padr % (ACC_ROWS - N)])
    src3 = srcs.reshape(NW, NPB, EPB)
    dst3 = dsts.reshape(NW, NPB, EPB)
    idx4 = jnp.stack([src3, dst3], axis=2)
    ones128 = jnp.ones((EPB, 128), jnp.float32)
    zeros128 = jnp.zeros((ZB, 128), jnp.float32)

    deg = _make_deg()(dst3, ones128, zeros128)
    dinv = _dinv(deg)

    convs = [("conv1", "mlp1", 1, 128), ("conv2", "mlp2", 1, 128),
             ("conv3", "mlp3", 1, 128), ("conv4", "mlp4", 2, 128),
             ("conv5", "mlp5", 4, 128)]

    y, st, gam, bet = pos, None, None, None
    for cname, mname, G, C in convs:
        cw = params[cname]["W"]
        cb = params[cname]["b"]
        layers = params[mname]
        w1 = layers[0]["W"]
        if st is None:
            cw = jnp.pad(cw, ((0, 0), (0, C - cw.shape[1])))
            cb = jnp.pad(cb, (0, C - cb.shape[0]))
            w1 = jnp.pad(w1, ((0, C - w1.shape[0]), (0, 0)))
            hp = _conv1_pre(pos, cw, dinv)
        else:
            hp = _conv_pre(y, st, gam, bet, cw, dinv, G, C)
        zeros_c = jnp.zeros((ZB, C), jnp.float32)
        sagg = _make_prop(G, C)(idx4, zeros_c, hp)
        y, st = _mlp_first(sagg, dinv, _row(cb), w1,
                           _row(layers[0]["b"]), G, C)
        gam, bet = _row(layers[0]["gamma"]), _row(layers[0]["beta"])
        for l in layers[1:]:
            y, st = _mlp_mid(y, st, gam, bet, l["W"], _row(l["b"]))
            gam, bet = _row(l["gamma"]), _row(l["beta"])

    p1, p2, p3 = params["lin1"], params["lin2"], params["lin3"]
    return _head(y, st, gam, bet, p1["W"], _row(p1["b"]),
                 p2["W"], _row(p2["b"]), p3["W"], _row(p3["b"]))
